# baseline (speedup 1.0000x reference)
import functools

import jax
import jax.numpy as jnp
from jax import lax
from jax.experimental import pallas as pl
from jax.experimental.pallas import tpu as pltpu
from jax.experimental.pallas import tpu_sc as plsc

NUM_USERS = 60000
NUM_ITEMS = 40000
N = NUM_USERS + NUM_ITEMS
D = 32
E = 1600000
L = 3

NC = 2
NS = 16
LANES = 16

NHALF = N // NC
DUMP = NHALF
ACC_ROWS = 50048
ZROWS = ACC_ROWS // NS
FLUSH = 3128
FLUSH_LAST = NHALF - 15 * FLUSH

W = 128
EROWS = E // W
CROWS = 3
CHUNK = W * CROWS
EROWS_PAD = 12576
CHUNKS_PER_TILE = EROWS_PAD // (NS * CROWS)
GROUPS = W // LANES


def _sc_sparse(ego, cols, vals, rows, zeros):
    mesh = plsc.VectorSubcoreMesh(
        core_axis_name="c", subcore_axis_name="s",
        num_cores=NC, num_subcores=NS)

    @functools.partial(
        pl.kernel,
        out_type=jax.ShapeDtypeStruct((N, D), jnp.float32),
        mesh=mesh,
        scratch_types=[
            [pltpu.VMEM((CROWS, W), jnp.int32) for _ in range(2)],
            [pltpu.VMEM((CROWS, W), jnp.float32) for _ in range(2)],
            [pltpu.VMEM((CROWS, W), jnp.int32) for _ in range(2)],
            [pltpu.VMEM((CROWS, W), jnp.int32) for _ in range(2)],
            [pltpu.VMEM((CHUNK, D), jnp.float32) for _ in range(2)],
            [pltpu.SemaphoreType.DMA for _ in range(2)],
            [pltpu.SemaphoreType.DMA for _ in range(2)],
            [pltpu.SemaphoreType.DMA for _ in range(2)],
            pltpu.VMEM_SHARED((ACC_ROWS, D), jnp.float32),
        ],
        compiler_params=pltpu.CompilerParams(
            needs_layout_passes=False, use_tc_tiling_on_sc=False),
    )
    def k(ego_h, cols_h, vals_h, rows_h, zeros_h, out_h,
          cols_v, vals_v, dst_v, scidx_v, g_v,
          sem_idx, sem_g, sem_sc, acc):
        c = lax.axis_index("c")
        s = lax.axis_index("s")
        base_node = c * NHALF
        pltpu.sync_copy(zeros_h, acc.at[pl.ds(s * ZROWS, ZROWS)])
        plsc.subcore_barrier()

        iota = lax.iota(jnp.int32, LANES)
        tile_row0 = s * CHUNKS_PER_TILE * CROWS

        def fire_idx(ch, p):
            row0 = tile_row0 + ch * CROWS
            pltpu.async_copy(cols_h.at[pl.ds(row0, CROWS)], cols_v[p],
                             sem_idx[p])
            pltpu.async_copy(vals_h.at[pl.ds(row0, CROWS)], vals_v[p],
                             sem_idx[p])
            pltpu.async_copy(rows_h.at[pl.ds(row0, CROWS)], dst_v[p],
                             sem_idx[p])

        def wait_idx(p):
            pltpu.make_async_copy(cols_h.at[pl.ds(0, CROWS)], cols_v[p],
                                  sem_idx[p]).wait()
            pltpu.make_async_copy(vals_h.at[pl.ds(0, CROWS)], vals_v[p],
                                  sem_idx[p]).wait()
            pltpu.make_async_copy(rows_h.at[pl.ds(0, CROWS)], dst_v[p],
                                  sem_idx[p]).wait()

        def drain_scatter(p):
            for j in range(CROWS):
                pltpu.make_async_copy(g_v[p].at[pl.ds(j * W, W)],
                                      acc.at[scidx_v[p].at[j]],
                                      sem_sc[p]).wait()

        dump_spread = DUMP + iota

        fire_idx(0, 0)

        @pl.loop(0, CHUNKS_PER_TILE // 2)
        def _outer(k_):
            for p in (0, 1):
                ch = k_ * 2 + p

                @pl.when(k_ >= 1)
                def _drain():
                    drain_scatter(p)

                wait_idx(p)

                gathers = [
                    pltpu.async_copy(ego_h.at[cols_v[p].at[j]],
                                     g_v[p].at[pl.ds(j * W, W)], sem_g[p])
                    for j in range(CROWS)
                ]

                @pl.when(ch < CHUNKS_PER_TILE - 1)
                def _prefetch():
                    fire_idx(ch + 1, 1 - p)

                for desc in gathers:
                    desc.wait()

                for j in range(CROWS):
                    for gg in range(GROUPS):
                        sl = pl.ds(gg * LANES, LANES)
                        dv = dst_v[p][j, sl]
                        dl = dv - base_node
                        ok = (dl >= 0) & (dl < NHALF)
                        dsp = dump_spread + 16 * ((j * GROUPS + gg) % 3)
                        scidx_v[p][j, sl] = jnp.where(ok, dl, dsp)
                        vv = vals_v[p][j, sl]
                        for i in range(LANES):
                            e = j * W + gg * LANES + i
                            bi = vv[jnp.full((LANES,), i, jnp.int32)]
                            h0 = g_v[p][e, pl.ds(0, LANES)]
                            g_v[p][e, pl.ds(0, LANES)] = h0 * bi
                            h1 = g_v[p][e, pl.ds(LANES, LANES)]
                            g_v[p][e, pl.ds(LANES, LANES)] = h1 * bi

                for j in range(CROWS):
                    pltpu.async_copy(g_v[p].at[pl.ds(j * W, W)],
                                     acc.at[scidx_v[p].at[j]], sem_sc[p],
                                     add=True)

        for p in (0, 1):
            drain_scatter(p)

        plsc.subcore_barrier()

        @pl.when(s < NS - 1)
        def _flush_main():
            pltpu.sync_copy(acc.at[pl.ds(s * FLUSH, FLUSH)],
                            out_h.at[pl.ds(c * NHALF + s * FLUSH, FLUSH)])

        @pl.when(s == NS - 1)
        def _flush_last():
            pltpu.sync_copy(
                acc.at[pl.ds(s * FLUSH, FLUSH_LAST)],
                out_h.at[pl.ds(c * NHALF + s * FLUSH, FLUSH_LAST)])

    return k(ego, cols, vals, rows, zeros)


RB = 1000


def _dense_layer(side, ego, acc, wg, wb, bg, bb, fin):
    def body(side_ref, ego_ref, acc_ref, wg_ref, wb_ref, bg_ref, bb_ref,
             eo_ref, ao_ref):
        sv = side_ref[...]
        ev = ego_ref[...]
        x = jnp.dot(sv, wg_ref[...], preferred_element_type=jnp.float32)
        x = x + bg_ref[...]
        x = x + jnp.dot(ev * sv, wb_ref[...], preferred_element_type=jnp.float32)
        x = x + bb_ref[...]
        x = jnp.where(x >= 0, x, jnp.float32(0.2) * x)
        nrm = jnp.sqrt(jnp.sum(x * x, axis=1, keepdims=True))
        x = x / jnp.maximum(nrm, jnp.float32(1e-12))
        eo_ref[...] = x
        ao_ref[...] = (acc_ref[...] + x) * jnp.float32(fin)

    bs_big = pl.BlockSpec((RB, D), lambda i: (i, 0))
    bs_w = pl.BlockSpec((D, D), lambda i: (0, 0))
    bs_b = pl.BlockSpec((1, D), lambda i: (0, 0))
    return pl.pallas_call(
        body,
        grid=(N // RB,),
        in_specs=[bs_big, bs_big, bs_big, bs_w, bs_w, bs_b, bs_b],
        out_specs=[bs_big, bs_big],
        out_shape=[jax.ShapeDtypeStruct((N, D), jnp.float32)] * 2,
    )(side, ego, acc, wg, wb, bg, bb)


def kernel(user_emb, item_emb, adj_rows, adj_cols, adj_vals, W_gc, W_bi,
           b_gc, b_bi):
    ego = jnp.concatenate([user_emb, item_emb], axis=0).astype(jnp.float32)
    cols2 = jnp.reshape(adj_cols.astype(jnp.int32), (EROWS, W))
    rows2 = jnp.reshape(adj_rows.astype(jnp.int32), (EROWS, W))
    vals2 = jnp.reshape(adj_vals.astype(jnp.float32), (EROWS, W))
    pad = EROWS_PAD - EROWS
    cols2 = jnp.pad(cols2, ((0, pad), (0, 0)))
    rows2 = jnp.pad(rows2, ((0, pad), (0, 0)), constant_values=N)
    vals2 = jnp.pad(vals2, ((0, pad), (0, 0)))
    zeros = jnp.zeros((ZROWS, D), jnp.float32)

    acc = ego
    for k in range(L):
        side = _sc_sparse(ego, cols2, vals2, rows2, zeros)
        fin = 0.25 if k == L - 1 else 1.0
        ego, acc = _dense_layer(
            side, ego, acc, W_gc[k].astype(jnp.float32),
            W_bi[k].astype(jnp.float32),
            b_gc[k].reshape(1, D).astype(jnp.float32),
            b_bi[k].reshape(1, D).astype(jnp.float32), fin)
    return acc[:NUM_USERS], acc[NUM_USERS:]


# speedup vs baseline: 1.1054x; 1.1054x over previous
import functools

import jax
import jax.numpy as jnp
from jax import lax
from jax.experimental import pallas as pl
from jax.experimental.pallas import tpu as pltpu
from jax.experimental.pallas import tpu_sc as plsc

NUM_USERS = 60000
NUM_ITEMS = 40000
N = NUM_USERS + NUM_ITEMS
D = 32
E = 1600000
L = 3

NC = 2
NS = 16
LANES = 16

NHALF = N // NC
DUMP = NHALF
ACC_ROWS = 50048
ZROWS = ACC_ROWS // NS
FLUSH = 3128
FLUSH_LAST = NHALF - 15 * FLUSH

W = 128
EROWS = E // W
CROWS = 2
CHUNK = W * CROWS
EROWS_PAD = 12544
CHUNKS_PER_TILE = EROWS_PAD // (NS * CROWS)
GROUPS = W // LANES


def _sc_sparse(ego, cols, vals, rows, zeros):
    mesh = plsc.VectorSubcoreMesh(
        core_axis_name="c", subcore_axis_name="s",
        num_cores=NC, num_subcores=NS)

    @functools.partial(
        pl.kernel,
        out_type=jax.ShapeDtypeStruct((N, D), jnp.float32),
        mesh=mesh,
        scratch_types=[
            [pltpu.VMEM((CROWS, W), jnp.int32) for _ in range(2)],
            [pltpu.VMEM((CROWS, W), jnp.float32) for _ in range(2)],
            [pltpu.VMEM((CROWS, W), jnp.int32) for _ in range(2)],
            [pltpu.VMEM((CROWS, W), jnp.int32) for _ in range(2)],
            [pltpu.VMEM((CHUNK, D), jnp.float32) for _ in range(2)],
            [pltpu.SemaphoreType.DMA for _ in range(2)],
            [pltpu.SemaphoreType.DMA for _ in range(2)],
            [pltpu.SemaphoreType.DMA for _ in range(2)],
            pltpu.VMEM_SHARED((ACC_ROWS, D), jnp.float32),
        ],
        compiler_params=pltpu.CompilerParams(
            needs_layout_passes=False, use_tc_tiling_on_sc=False),
    )
    def k(ego_h, cols_h, vals_h, rows_h, zeros_h, out_h,
          cols_v, vals_v, dst_v, scidx_v, g_v,
          sem_idx, sem_g, sem_sc, acc):
        c = lax.axis_index("c")
        s = lax.axis_index("s")
        base_node = c * NHALF
        pltpu.sync_copy(zeros_h, acc.at[pl.ds(s * ZROWS, ZROWS)])
        plsc.subcore_barrier()

        iota = lax.iota(jnp.int32, LANES)
        tile_row0 = s * CHUNKS_PER_TILE * CROWS

        def fire_idx(ch, p):
            row0 = tile_row0 + ch * CROWS
            pltpu.async_copy(cols_h.at[pl.ds(row0, CROWS)], cols_v[p],
                             sem_idx[p])
            pltpu.async_copy(vals_h.at[pl.ds(row0, CROWS)], vals_v[p],
                             sem_idx[p])
            pltpu.async_copy(rows_h.at[pl.ds(row0, CROWS)], dst_v[p],
                             sem_idx[p])

        def wait_idx(p):
            pltpu.make_async_copy(cols_h.at[pl.ds(0, CROWS)], cols_v[p],
                                  sem_idx[p]).wait()
            pltpu.make_async_copy(vals_h.at[pl.ds(0, CROWS)], vals_v[p],
                                  sem_idx[p]).wait()
            pltpu.make_async_copy(rows_h.at[pl.ds(0, CROWS)], dst_v[p],
                                  sem_idx[p]).wait()

        def drain_scatter(p):
            for j in range(CROWS):
                pltpu.make_async_copy(g_v[p].at[pl.ds(j * W, W)],
                                      acc.at[scidx_v[p].at[j]],
                                      sem_sc[p]).wait()

        dump_spread = DUMP + iota

        fire_idx(0, 0)

        @pl.loop(0, CHUNKS_PER_TILE // 2)
        def _outer(k_):
            for p in (0, 1):
                ch = k_ * 2 + p

                @pl.when(k_ >= 1)
                def _drain():
                    drain_scatter(p)

                wait_idx(p)

                gathers = [
                    pltpu.async_copy(ego_h.at[cols_v[p].at[j]],
                                     g_v[p].at[pl.ds(j * W, W)], sem_g[p])
                    for j in range(CROWS)
                ]

                @pl.when(ch < CHUNKS_PER_TILE - 1)
                def _prefetch():
                    fire_idx(ch + 1, 1 - p)

                for desc in gathers:
                    desc.wait()

                for j in range(CROWS):
                    for gg in range(GROUPS):
                        sl = pl.ds(gg * LANES, LANES)
                        dv = dst_v[p][j, sl]
                        dl = dv - base_node
                        ok = (dl >= 0) & (dl < NHALF)
                        dsp = dump_spread + 16 * (gg & 1)
                        scidx_v[p][j, sl] = jnp.where(ok, dl, dsp)
                        vv = vals_v[p][j, sl]
                        for i in range(LANES):
                            e = j * W + gg * LANES + i
                            bi = vv[jnp.full((LANES,), i, jnp.int32)]
                            h0 = g_v[p][e, pl.ds(0, LANES)]
                            g_v[p][e, pl.ds(0, LANES)] = h0 * bi
                            h1 = g_v[p][e, pl.ds(LANES, LANES)]
                            g_v[p][e, pl.ds(LANES, LANES)] = h1 * bi

                for j in range(CROWS):
                    pltpu.async_copy(g_v[p].at[pl.ds(j * W, W)],
                                     acc.at[scidx_v[p].at[j]], sem_sc[p],
                                     add=True)

        for p in (0, 1):
            drain_scatter(p)

        plsc.subcore_barrier()

        @pl.when(s < NS - 1)
        def _flush_main():
            pltpu.sync_copy(acc.at[pl.ds(s * FLUSH, FLUSH)],
                            out_h.at[pl.ds(c * NHALF + s * FLUSH, FLUSH)])

        @pl.when(s == NS - 1)
        def _flush_last():
            pltpu.sync_copy(
                acc.at[pl.ds(s * FLUSH, FLUSH_LAST)],
                out_h.at[pl.ds(c * NHALF + s * FLUSH, FLUSH_LAST)])

    return k(ego, cols, vals, rows, zeros)


RB = 1000


def _dense_layer(side, ego, acc, wg, wb, bg, bb, fin):
    def body(side_ref, ego_ref, acc_ref, wg_ref, wb_ref, bg_ref, bb_ref,
             eo_ref, ao_ref):
        sv = side_ref[...]
        ev = ego_ref[...]
        x = jnp.dot(sv, wg_ref[...], preferred_element_type=jnp.float32)
        x = x + bg_ref[...]
        x = x + jnp.dot(ev * sv, wb_ref[...], preferred_element_type=jnp.float32)
        x = x + bb_ref[...]
        x = jnp.where(x >= 0, x, jnp.float32(0.2) * x)
        nrm = jnp.sqrt(jnp.sum(x * x, axis=1, keepdims=True))
        x = x / jnp.maximum(nrm, jnp.float32(1e-12))
        eo_ref[...] = x
        ao_ref[...] = (acc_ref[...] + x) * jnp.float32(fin)

    bs_big = pl.BlockSpec((RB, D), lambda i: (i, 0))
    bs_w = pl.BlockSpec((D, D), lambda i: (0, 0))
    bs_b = pl.BlockSpec((1, D), lambda i: (0, 0))
    return pl.pallas_call(
        body,
        grid=(N // RB,),
        in_specs=[bs_big, bs_big, bs_big, bs_w, bs_w, bs_b, bs_b],
        out_specs=[bs_big, bs_big],
        out_shape=[jax.ShapeDtypeStruct((N, D), jnp.float32)] * 2,
    )(side, ego, acc, wg, wb, bg, bb)


def kernel(user_emb, item_emb, adj_rows, adj_cols, adj_vals, W_gc, W_bi,
           b_gc, b_bi):
    ego = jnp.concatenate([user_emb, item_emb], axis=0).astype(jnp.float32)
    cols2 = jnp.reshape(adj_cols.astype(jnp.int32), (EROWS, W))
    rows2 = jnp.reshape(adj_rows.astype(jnp.int32), (EROWS, W))
    vals2 = jnp.reshape(adj_vals.astype(jnp.float32), (EROWS, W))
    pad = EROWS_PAD - EROWS
    cols2 = jnp.pad(cols2, ((0, pad), (0, 0)))
    rows2 = jnp.pad(rows2, ((0, pad), (0, 0)), constant_values=N)
    vals2 = jnp.pad(vals2, ((0, pad), (0, 0)))
    zeros = jnp.zeros((ZROWS, D), jnp.float32)

    acc = ego
    for k in range(L):
        side = _sc_sparse(ego, cols2, vals2, rows2, zeros)
        fin = 0.25 if k == L - 1 else 1.0
        ego, acc = _dense_layer(
            side, ego, acc, W_gc[k].astype(jnp.float32),
            W_bi[k].astype(jnp.float32),
            b_gc[k].reshape(1, D).astype(jnp.float32),
            b_bi[k].reshape(1, D).astype(jnp.float32), fin)
    return acc[:NUM_USERS], acc[NUM_USERS:]
